# baseline (speedup 1.0000x reference)
import functools

import jax
import jax.numpy as jnp
from jax import lax
from jax.experimental import pallas as pl
from jax.experimental.pallas import tpu as pltpu
from jax.experimental.pallas import tpu_sc as plsc

_B = 16384
_D = 32
_LANES = 16
_NC = 2
_NS = 16
_NW = _NC * _NS
_BPW = _B // _NW
_CHUNK = 128
_NCHUNK = _BPW // _CHUNK
_GROUPS = _BPW // _LANES


_TPAD = _LANES + 1


def _dot_body(uids, iids, utab, itab, out,
              uidx_v, iidx_v, urows_v, irows_v, out_v, tsc_v, sem):
    wid = lax.axis_index("s") * _NC + lax.axis_index("c")
    base = wid * _NCHUNK
    pltpu.sync_copy(uids.at[pl.ds(base, _NCHUNK)], uidx_v)
    pltpu.sync_copy(iids.at[pl.ds(base, _NCHUNK)], iidx_v)
    copies = []
    for j in range(_NCHUNK):
        copies.append(pltpu.async_copy(
            utab.at[uidx_v.at[j]], urows_v.at[pl.ds(j * _CHUNK, _CHUNK)], sem))
        copies.append(pltpu.async_copy(
            itab.at[iidx_v.at[j]], irows_v.at[pl.ds(j * _CHUNK, _CHUNK)], sem))
    for c in copies:
        c.wait()

    lane = lax.iota(jnp.int32, _LANES)

    def group(g, carry):
        rbase = g * _LANES
        for r in range(_LANES):
            row = rbase + r
            u0 = urows_v[row, pl.ds(0, _LANES)]
            u1 = urows_v[row, pl.ds(_LANES, _LANES)]
            v0 = irows_v[row, pl.ds(0, _LANES)]
            v1 = irows_v[row, pl.ds(_LANES, _LANES)]
            plsc.store_scatter(tsc_v, [lane * _TPAD + r], u0 * v0 + u1 * v1)
        res = tsc_v[pl.ds(0, _LANES)]
        for l in range(1, _LANES):
            res = res + tsc_v[pl.ds(l * _TPAD, _LANES)]
        out_v[pl.ds(rbase, _LANES)] = res
        return carry

    lax.fori_loop(0, _GROUPS, group, 0)
    pltpu.sync_copy(out_v, out.at[pl.ds(wid * _BPW, _BPW)])


def kernel(user_ids, item_ids, user_table, item_table):
    uids = user_ids.astype(jnp.int32).reshape(_NW * _NCHUNK, _CHUNK)
    iids = item_ids.astype(jnp.int32).reshape(_NW * _NCHUNK, _CHUNK)
    mesh = plsc.VectorSubcoreMesh(core_axis_name="c", subcore_axis_name="s")
    f = pl.kernel(
        _dot_body,
        mesh=mesh,
        compiler_params=pltpu.CompilerParams(
            needs_layout_passes=False, use_tc_tiling_on_sc=False),
        out_type=jax.ShapeDtypeStruct((_B,), jnp.float32),
        scratch_types=[
            pltpu.VMEM((_NCHUNK, _CHUNK), jnp.int32),
            pltpu.VMEM((_NCHUNK, _CHUNK), jnp.int32),
            pltpu.VMEM((_BPW, _D), jnp.float32),
            pltpu.VMEM((_BPW, _D), jnp.float32),
            pltpu.VMEM((_BPW,), jnp.float32),
            pltpu.VMEM((_LANES * _TPAD,), jnp.float32),
            pltpu.SemaphoreType.DMA,
        ],
    )
    return f(uids, iids, user_table, item_table)


# speedup vs baseline: 1.5017x; 1.5017x over previous
import functools

import jax
import jax.numpy as jnp
from jax import lax
from jax.experimental import pallas as pl
from jax.experimental.pallas import tpu as pltpu
from jax.experimental.pallas import tpu_sc as plsc

_B = 16384
_D = 32
_LANES = 16
_NC = 2
_NS = 16
_NW = _NC * _NS
_BPW = _B // _NW
_CHUNK = 128
_NCHUNK = _BPW // _CHUNK
_GROUPS = _CHUNK // _LANES
_NSEM = 8
_TPAD = _LANES + 1


def _dot_body(uids, iids, utab, itab, out,
              uidx_v, iidx_v,
              ubuf0, ubuf1, ibuf0, ibuf1,
              out_v, tsc_v, *sems):
    wid = lax.axis_index("s") * _NC + lax.axis_index("c")
    base = wid * _BPW
    pltpu.sync_copy(uids.at[pl.ds(base, _BPW)], uidx_v)
    pltpu.sync_copy(iids.at[pl.ds(base, _BPW)], iidx_v)

    ubufs = (ubuf0, ubuf1)
    ibufs = (ibuf0, ibuf1)
    lane = lax.iota(jnp.int32, _LANES)

    def issue(j, p):
        ub, ib = ubufs[p], ibufs[p]

        def body(k, carry):
            uvec = uidx_v[pl.ds(j * _CHUNK + k * _LANES, _LANES)]
            ivec = iidx_v[pl.ds(j * _CHUNK + k * _LANES, _LANES)]
            for r in range(_LANES):
                rr = k * _LANES + r
                sem = sems[r % _NSEM]
                pltpu.async_copy(
                    utab.at[pl.ds(uvec[r], 1)], ub.at[pl.ds(rr, 1)], sem)
                pltpu.async_copy(
                    itab.at[pl.ds(ivec[r], 1)], ib.at[pl.ds(rr, 1)], sem)
            return carry

        lax.fori_loop(0, _GROUPS, body, 0)

    def drain(p):
        nrows = 2 * _CHUNK // _NSEM
        for s in range(_NSEM):
            pltpu.make_async_copy(
                utab.at[pl.ds(0, nrows)], ubufs[p].at[pl.ds(0, nrows)],
                sems[s]).wait()

    def compute(j, p):
        ub, ib = ubufs[p], ibufs[p]

        def group(g, carry):
            rbase = g * _LANES
            for r in range(_LANES):
                row = rbase + r
                u0 = ub[row, pl.ds(0, _LANES)]
                u1 = ub[row, pl.ds(_LANES, _LANES)]
                v0 = ib[row, pl.ds(0, _LANES)]
                v1 = ib[row, pl.ds(_LANES, _LANES)]
                plsc.store_scatter(tsc_v, [lane * _TPAD + r], u0 * v0 + u1 * v1)
            res = tsc_v[pl.ds(0, _LANES)]
            for l in range(1, _LANES):
                res = res + tsc_v[pl.ds(l * _TPAD, _LANES)]
            out_v[pl.ds(j * _CHUNK + rbase, _LANES)] = res
            return carry

        lax.fori_loop(0, _GROUPS, group, 0)

    issue(0, 0)
    for j in range(_NCHUNK):
        if j + 1 < _NCHUNK:
            issue(j + 1, (j + 1) % 2)
        drain(j % 2)
        compute(j, j % 2)
    pltpu.sync_copy(out_v, out.at[pl.ds(base, _BPW)])


def kernel(user_ids, item_ids, user_table, item_table):
    uids = user_ids.astype(jnp.int32)
    iids = item_ids.astype(jnp.int32)
    mesh = plsc.VectorSubcoreMesh(core_axis_name="c", subcore_axis_name="s")
    f = pl.kernel(
        _dot_body,
        mesh=mesh,
        compiler_params=pltpu.CompilerParams(needs_layout_passes=False),
        out_type=jax.ShapeDtypeStruct((_B,), jnp.float32),
        scratch_types=[
            pltpu.VMEM((_BPW,), jnp.int32),
            pltpu.VMEM((_BPW,), jnp.int32),
            pltpu.VMEM((_CHUNK, _D), jnp.float32),
            pltpu.VMEM((_CHUNK, _D), jnp.float32),
            pltpu.VMEM((_CHUNK, _D), jnp.float32),
            pltpu.VMEM((_CHUNK, _D), jnp.float32),
            pltpu.VMEM((_BPW,), jnp.float32),
            pltpu.VMEM((_LANES * _TPAD,), jnp.float32),
        ] + [pltpu.SemaphoreType.DMA] * _NSEM,
    )
    return f(uids, iids, user_table, item_table)
